# Initial kernel scaffold; baseline (speedup 1.0000x reference)
import jax
import jax.numpy as jnp
from jax.experimental import pallas as pl


def kernel(pos, x, Wh, bh, gh, beh, Wr, br, gr, ber, W1, b1, g1, be1, W2, b2, g2, be2, W3, b3):
    raise NotImplementedError("write your pallas kernel here")


# speedup vs baseline: 18.5692x; 18.5692x over previous
import functools

import numpy as np
import jax
import jax.numpy as jnp
from jax import lax
from jax.experimental import pallas as pl
from jax.experimental.pallas import tpu as pltpu
from jax.experimental.pallas import tpu_sc as plsc

_N = 10000
_NP = 10240
_K = 16
_BNC = np.float32(np.sqrt(1.0 + 1e-5))


def _knn_body(xt_ref, xtt_ref, idx_ref, *, n_valid, k):
    q = xt_ref[...]
    xtt = xtt_ref[...]
    inner = -2.0 * lax.dot(q, xtt, preferred_element_type=jnp.float32)
    sq_r = jnp.sum(q * q, axis=1)
    sq_c = jnp.sum(xtt * xtt, axis=0)
    s = -((sq_r[:, None] + inner) + sq_c[None, :])
    col = lax.broadcasted_iota(jnp.int32, s.shape, 1)
    s = jnp.where(col < n_valid, s, -jnp.inf)
    picks = []
    for _ in range(k):
        m = jnp.max(s, axis=1)
        c = jnp.min(jnp.where(s == m[:, None], col, jnp.int32(2**30)), axis=1)
        picks.append(c)
        s = jnp.where(col == c[:, None], -jnp.inf, s)
    idx_ref[...] = jnp.stack(picks, axis=1)


def _knn(xt_pad, xtt_pad, k=_K):
    npad, cp = xt_pad.shape
    return pl.pallas_call(
        functools.partial(_knn_body, n_valid=_N, k=k),
        grid=(npad // 128,),
        in_specs=[
            pl.BlockSpec((128, cp), lambda i: (i, 0)),
            pl.BlockSpec((cp, npad), lambda i: (0, 0)),
        ],
        out_specs=pl.BlockSpec((128, k), lambda i: (i, 0)),
        out_shape=jax.ShapeDtypeStruct((npad, k), jnp.int32),
    )(xt_pad, xtt_pad)


def _sc_gather_cat(table, idx_flat):
    info = plsc.get_sparse_core_info()
    nc, ns = info.num_cores, info.num_subcores
    nw = nc * ns
    total_e = _NP * _K
    epw = total_e // nw
    che = 128
    chunks = epw // che
    npw = epw // _K
    npc = che // _K
    assert epw * nw == total_e and chunks * che == epw and npc * _K == che

    idx3 = idx_flat.reshape(nw, chunks, che)

    @functools.partial(
        pl.kernel,
        mesh=plsc.VectorSubcoreMesh(core_axis_name="c", subcore_axis_name="s"),
        out_type=jax.ShapeDtypeStruct((total_e, 128), jnp.float32),
        scratch_types=[
            pltpu.VMEM((chunks, che), jnp.int32),
            pltpu.VMEM((npw, 128), jnp.float32),
            pltpu.VMEM((che, 128), jnp.float32),
            pltpu.VMEM((che, 128), jnp.float32),
            pltpu.SemaphoreType.DMA,
        ],
    )
    def sck(tab_hbm, idx_hbm, cat_hbm, idx_v, own_v, rows_v, cat_v, sem):
        wid = lax.axis_index("s") * nc + lax.axis_index("c")
        base_n = wid * npw
        base_e = wid * epw
        pltpu.sync_copy(idx_hbm.at[wid], idx_v)
        pltpu.sync_copy(tab_hbm.at[pl.ds(base_n, npw)], own_v)

        def chunk_body(c, carry):
            pltpu.async_copy(tab_hbm.at[idx_v.at[c]], rows_v, sem).wait()
            for i in range(npc):
                for ch in range(4):
                    sl = pl.ds(ch * 16, 16)
                    sh = pl.ds(64 + ch * 16, 16)
                    xi = own_v[c * npc + i, sl]
                    for j in range(_K):
                        e = i * _K + j
                        cat_v[e, sl] = xi
                        cat_v[e, sh] = rows_v[e, sl] - xi
            pltpu.sync_copy(cat_v, cat_hbm.at[pl.ds(base_e + c * che, che)])
            return carry

        lax.fori_loop(0, chunks, chunk_body, 0)

    return sck(table, idx3)


def _econv_res_body(cat_ref, w_ref, b_ref, g_ref, be_ref, hres_ref, o_ref):
    y = lax.dot(cat_ref[...], w_ref[...], preferred_element_type=jnp.float32)
    y = ((y + b_ref[...]) / _BNC) * g_ref[...] + be_ref[...]
    y = jnp.maximum(y, 0.0)
    y = jnp.max(y.reshape(-1, _K, 64), axis=1)
    o_ref[...] = y + hres_ref[...]


def _econv_body(cat_ref, w_ref, b_ref, g_ref, be_ref, o_ref):
    y = lax.dot(cat_ref[...], w_ref[...], preferred_element_type=jnp.float32)
    y = ((y + b_ref[...]) / _BNC) * g_ref[...] + be_ref[...]
    y = jnp.maximum(y, 0.0)
    o_ref[...] = jnp.max(y.reshape(-1, _K, 64), axis=1)


def _econv(cat, wt, bvec, gvec, bevec, hres=None, be=512):
    etot = cat.shape[0]
    bn = be // _K
    spec_v = pl.BlockSpec((1, 64), lambda i: (0, 0))
    in_specs = [
        pl.BlockSpec((be, 128), lambda i: (i, 0)),
        pl.BlockSpec((128, 64), lambda i: (0, 0)),
        spec_v, spec_v, spec_v,
    ]
    args = [cat, wt, bvec, gvec, bevec]
    body = _econv_body
    if hres is not None:
        in_specs.append(pl.BlockSpec((bn, 64), lambda i: (i, 0)))
        args.append(hres)
        body = _econv_res_body
    return pl.pallas_call(
        body,
        grid=(etot // be,),
        in_specs=in_specs,
        out_specs=pl.BlockSpec((bn, 64), lambda i: (i, 0)),
        out_shape=jax.ShapeDtypeStruct((etot // _K, 64), jnp.float32),
    )(*args)


def _mlp_body(h_ref, w1_ref, b1_ref, g1_ref, be1_ref,
              w2_ref, b2_ref, g2_ref, be2_ref, w3_ref, b3_ref, p_ref):
    y = lax.dot(h_ref[...], w1_ref[...], preferred_element_type=jnp.float32)
    y = ((y + b1_ref[...]) / _BNC) * g1_ref[...] + be1_ref[...]
    y = jnp.maximum(y, 0.0)
    y = lax.dot(y, w2_ref[...], preferred_element_type=jnp.float32)
    y = ((y + b2_ref[...]) / _BNC) * g2_ref[...] + be2_ref[...]
    y = jnp.maximum(y, 0.0)
    p_ref[...] = (lax.dot(y, w3_ref[...], preferred_element_type=jnp.float32)
                  + b3_ref[...])


def _mlp(h, w1t, b1v, g1v, be1v, w2t, b2v, g2v, be2v, w3t, b3v, bm=512):
    npad = h.shape[0]
    spec_v64 = pl.BlockSpec((1, 64), lambda i: (0, 0))
    spec_v32 = pl.BlockSpec((1, 32), lambda i: (0, 0))
    spec_v13 = pl.BlockSpec((1, 13), lambda i: (0, 0))
    return pl.pallas_call(
        _mlp_body,
        grid=(npad // bm,),
        in_specs=[
            pl.BlockSpec((bm, 64), lambda i: (i, 0)),
            pl.BlockSpec((64, 64), lambda i: (0, 0)),
            spec_v64, spec_v64, spec_v64,
            pl.BlockSpec((64, 32), lambda i: (0, 0)),
            spec_v32, spec_v32, spec_v32,
            pl.BlockSpec((32, 13), lambda i: (0, 0)),
            spec_v13,
        ],
        out_specs=pl.BlockSpec((bm, 13), lambda i: (i, 0)),
        out_shape=jax.ShapeDtypeStruct((npad, 13), jnp.float32),
    )(h, w1t, b1v, g1v, be1v, w2t, b2v, g2v, be2v, w3t, b3v)


def _padr(a, rows):
    return jnp.pad(a, ((0, rows - a.shape[0]), (0, 0)))


def kernel(pos, x, Wh, bh, gh, beh, Wr, br, gr, ber,
           W1, b1, g1, be1, W2, b2, g2, be2, W3, b3):
    del pos
    featT = x[0, :, :, 0].T
    cin = featT.shape[1]

    xt3 = _padr(featT[:, 0:3], _NP)
    xt3 = jnp.pad(xt3, ((0, 0), (0, 5)))
    idx0 = _knn(xt3, xt3.T)

    table1 = jnp.pad(_padr(featT, _NP), ((0, 0), (0, 128 - cin)))
    cat0 = _sc_gather_cat(table1, idx0.reshape(-1))
    w0t = jnp.zeros((128, 64), jnp.float32)
    w0t = w0t.at[0:cin].set(Wh[:, :cin].T)
    w0t = w0t.at[64:64 + cin].set(Wh[:, cin:].T)
    h = _econv(cat0, w0t, bh.reshape(1, 64), gh.reshape(1, 64),
               beh.reshape(1, 64))

    idx1 = _knn(h, h.T)

    table2 = jnp.pad(h, ((0, 0), (0, 64)))
    cat1 = _sc_gather_cat(table2, idx1.reshape(-1))
    wrt = jnp.concatenate([Wr[:, :64].T, Wr[:, 64:].T], axis=0)
    h2 = _econv(cat1, wrt, br.reshape(1, 64), gr.reshape(1, 64),
                ber.reshape(1, 64), hres=h)

    p = _mlp(h2,
             W1.T, b1.reshape(1, 64), g1.reshape(1, 64), be1.reshape(1, 64),
             W2.T, b2.reshape(1, 32), g2.reshape(1, 32), be2.reshape(1, 32),
             W3.T, b3.reshape(1, 13))

    return p[:_N].T[None, :, :]
